# Initial kernel scaffold; baseline (speedup 1.0000x reference)
import jax
import jax.numpy as jnp
from jax.experimental import pallas as pl


def kernel(W, mask):
    raise NotImplementedError("write your pallas kernel here")


# speedup vs baseline: 11.3009x; 11.3009x over previous
import functools

import jax
import jax.numpy as jnp
from jax import lax
from jax.experimental import pallas as pl
from jax.experimental.pallas import tpu as pltpu, tpu_sc as plsc

NEG = -1e20
LN2 = 0.6931471805599453
MU_WORDS = 129 * 129
MU_OUT = 16648


def _dtw_body(w_hbm, mask_hbm, mu_hbm, pi_hbm, w_v, mask_v, mu_v, pi_v):
    b = lax.axis_index("c") * 16 + lax.axis_index("s")
    pltpu.sync_copy(w_hbm.at[b], w_v)
    pltpu.sync_copy(mask_hbm.at[b], mask_v)
    iota = lax.iota(jnp.int32, 16)
    negv = jnp.full((16,), NEG, jnp.float32)

    def init_chunk(c, carry):
        row_idx = c * 16 + iota
        row_val = jnp.where(row_idx == 0, 0.0, negv)
        plsc.store_scatter(mu_v, [row_idx], row_val, mask=row_idx <= 128)
        col_i = c * 16 + iota + 1
        col_ic = jnp.minimum(col_i, 128)
        plsc.store_scatter(mu_v, [col_ic * 129], negv, mask=col_i <= 128)
        return carry

    lax.fori_loop(0, 9, init_chunk, 0)

    def diag_body(d, carry):
        il = jnp.maximum(1, d - 128)
        ih = jnp.minimum(128, d - 1)
        nch = (ih - il + 16) // 16

        def chunk_body(c, inner):
            i = il + c * 16 + iota
            ic = jnp.minimum(i, ih)
            base = ic * 128 + d
            up = plsc.load_gather(mu_v, [base - 129])
            lf = plsc.load_gather(mu_v, [base - 1])
            dg = plsc.load_gather(mu_v, [base - 130])
            w = plsc.load_gather(w_v, [ic * 127 + d - 129])
            m = jnp.maximum(jnp.maximum(up, lf), dg)
            s = (jnp.exp(jnp.maximum(up - m, -80.0))
                 + jnp.exp(jnp.maximum(lf - m, -80.0))
                 + jnp.exp(jnp.maximum(dg - m, -80.0)))
            big = s >= 1.5
            sm = jnp.where(big, s * 0.5, s)
            e = jnp.where(big, LN2, 0.0)
            t = (sm - 1.0) / (sm + 1.0)
            t2 = t * t
            p = 2.0 * t * (1.0 + t2 * (1.0 / 3.0 + t2 * (0.2 + t2 * (1.0 / 7.0))))
            plsc.store_scatter(mu_v, [base], m + e + p + w, mask=i <= ih)
            return inner

        lax.fori_loop(0, nch, chunk_body, 0)
        return carry

    lax.fori_loop(2, 257, diag_body, 0)

    def row_body(i, carry):
        rb = i * 129

        def pchunk(c, inner):
            j = c * 16 + iota + 1
            q = (i - 1) * 128 + (j - 1)
            up = plsc.load_gather(mu_v, [rb - 129 + j])
            lf = plsc.load_gather(mu_v, [rb + j - 1])
            dg = plsc.load_gather(mu_v, [rb - 130 + j])
            w = plsc.load_gather(w_v, [q])
            mk = plsc.load_gather(mask_v, [q])
            lu, ll, ld = up + w, lf + w, dg + w
            m = jnp.maximum(jnp.maximum(lu, ll), ld)
            eu = jnp.exp(jnp.maximum(lu - m, -80.0))
            el = jnp.exp(jnp.maximum(ll - m, -80.0))
            ed = jnp.exp(jnp.maximum(ld - m, -80.0))
            r = mk / (eu + el + ed)
            base3 = q * 3
            plsc.store_scatter(pi_v, [base3], eu * r)
            plsc.store_scatter(pi_v, [base3 + 1], el * r)
            plsc.store_scatter(pi_v, [base3 + 2], ed * r)
            return inner

        lax.fori_loop(0, 8, pchunk, 0)
        return carry

    lax.fori_loop(1, 129, row_body, 0)

    pltpu.sync_copy(mu_v, mu_hbm.at[b])
    pltpu.sync_copy(pi_v, pi_hbm.at[b])


@jax.jit
def _dtw_sc(w_flat, mask_flat):
    batch = w_flat.shape[0]
    mesh = plsc.VectorSubcoreMesh(core_axis_name="c", subcore_axis_name="s")
    f = pl.kernel(
        _dtw_body,
        out_type=(
            jax.ShapeDtypeStruct((batch, MU_OUT), jnp.float32),
            jax.ShapeDtypeStruct((batch, 128 * 128 * 3), jnp.float32),
        ),
        mesh=mesh,
        scratch_types=[
            pltpu.VMEM((128 * 128,), jnp.float32),
            pltpu.VMEM((128 * 128,), jnp.float32),
            pltpu.VMEM((MU_OUT,), jnp.float32),
            pltpu.VMEM((128 * 128 * 3,), jnp.float32),
        ],
        compiler_params=pltpu.CompilerParams(needs_layout_passes=False),
    )
    return f(w_flat, mask_flat)


def kernel(W, mask):
    batch, Na, Nb = W.shape
    w_flat = W.reshape(batch, Na * Nb)
    mask_flat = mask.reshape(batch, Na * Nb)
    mu_pad, pi_flat = _dtw_sc(w_flat, mask_flat)
    mu = mu_pad[:, :MU_WORDS].reshape(batch, Na + 1, Nb + 1)
    pi = pi_flat.reshape(batch, Na, Nb, 3)
    return mu, pi


# speedup vs baseline: 15.1830x; 1.3435x over previous
import functools

import jax
import jax.numpy as jnp
from jax import lax
from jax.experimental import pallas as pl
from jax.experimental.pallas import tpu as pltpu, tpu_sc as plsc

NEG = -1e20
LN2 = 0.6931471805599453
MU_WORDS = 129 * 129
MU_OUT = 16648


def _dtw_body(w_hbm, mask_hbm, mu_hbm, pi_hbm, w_v, mask_v, mu_v, pi_v):
    b = lax.axis_index("c") * 16 + lax.axis_index("s")
    pltpu.sync_copy(w_hbm.at[b], w_v)
    pltpu.sync_copy(mask_hbm.at[b], mask_v)
    iota = lax.iota(jnp.int32, 16)
    negv = jnp.full((16,), NEG, jnp.float32)

    def init_chunk(c, carry):
        row_idx = c * 16 + iota
        row_val = jnp.where(row_idx == 0, 0.0, negv)
        plsc.store_scatter(mu_v, [row_idx], row_val, mask=row_idx <= 128)
        col_i = c * 16 + iota + 1
        col_ic = jnp.minimum(col_i, 128)
        plsc.store_scatter(mu_v, [col_ic * 129], negv, mask=col_i <= 128)
        return carry

    lax.fori_loop(0, 9, init_chunk, 0)

    def diag_body(d, carry):
        il = jnp.maximum(1, d - 128)
        ih = jnp.minimum(128, d - 1)
        nch = (ih - il + 16) // 16

        @plsc.parallel_loop(0, nch)
        def chunk_body(c):
            i = il + c * 16 + iota
            ic = jnp.minimum(i, ih)
            valid = i <= ih
            base = ic * 128 + d
            wq = ic * 127 + d - 129
            up = plsc.load_gather(mu_v, [base - 129])
            lf = plsc.load_gather(mu_v, [base - 1])
            dg = plsc.load_gather(mu_v, [base - 130])
            w = plsc.load_gather(w_v, [wq])
            mk = plsc.load_gather(mask_v, [wq])
            m = jnp.maximum(jnp.maximum(up, lf), dg)
            eu = jnp.exp(jnp.maximum(up - m, -80.0))
            el = jnp.exp(jnp.maximum(lf - m, -80.0))
            ed = jnp.exp(jnp.maximum(dg - m, -80.0))
            s = eu + el + ed
            big = s >= 1.5
            sm = jnp.where(big, s * 0.5, s)
            e = jnp.where(big, LN2, 0.0)
            t = (sm - 1.0) / (sm + 1.0)
            t2 = t * t
            p = 2.0 * t * (1.0 + t2 * (1.0 / 3.0 + t2 * (0.2 + t2 * (1.0 / 7.0))))
            plsc.store_scatter(mu_v, [base], m + e + p + w, mask=valid)
            r = mk / s
            pib = wq * 3
            plsc.store_scatter(pi_v, [pib], eu * r, mask=valid)
            plsc.store_scatter(pi_v, [pib + 1], el * r, mask=valid)
            plsc.store_scatter(pi_v, [pib + 2], ed * r, mask=valid)

        return carry

    lax.fori_loop(2, 257, diag_body, 0)

    pltpu.sync_copy(mu_v, mu_hbm.at[b])
    pltpu.sync_copy(pi_v, pi_hbm.at[b])


@jax.jit
def _dtw_sc(w_flat, mask_flat):
    batch = w_flat.shape[0]
    mesh = plsc.VectorSubcoreMesh(core_axis_name="c", subcore_axis_name="s")
    f = pl.kernel(
        _dtw_body,
        out_type=(
            jax.ShapeDtypeStruct((batch, MU_OUT), jnp.float32),
            jax.ShapeDtypeStruct((batch, 128 * 128 * 3), jnp.float32),
        ),
        mesh=mesh,
        scratch_types=[
            pltpu.VMEM((128 * 128,), jnp.float32),
            pltpu.VMEM((128 * 128,), jnp.float32),
            pltpu.VMEM((MU_OUT,), jnp.float32),
            pltpu.VMEM((128 * 128 * 3,), jnp.float32),
        ],
        compiler_params=pltpu.CompilerParams(needs_layout_passes=False),
    )
    return f(w_flat, mask_flat)


def kernel(W, mask):
    batch, Na, Nb = W.shape
    w_flat = W.reshape(batch, Na * Nb)
    mask_flat = mask.reshape(batch, Na * Nb)
    mu_pad, pi_flat = _dtw_sc(w_flat, mask_flat)
    mu = mu_pad[:, :MU_WORDS].reshape(batch, Na + 1, Nb + 1)
    pi = pi_flat.reshape(batch, Na, Nb, 3)
    return mu, pi
